# Initial kernel scaffold; baseline (speedup 1.0000x reference)
import jax
import jax.numpy as jnp
from jax.experimental import pallas as pl


def kernel(x, w, gamma, beta):
    raise NotImplementedError("write your pallas kernel here")


# speedup vs baseline: 2.0253x; 2.0253x over previous
import functools

import jax
import jax.numpy as jnp
from jax.experimental import pallas as pl
from jax.experimental.pallas import tpu as pltpu


def _make_conv_stats_kernel(KH, KW, W_out):
    def _kernel(x_ref, w_ref, conv_ref, stats_ref, xcol_ref):
        _, Cin, HW = x_ref.shape
        xb = x_ref[0].astype(jnp.bfloat16)

        col = jax.lax.broadcasted_iota(jnp.int32, (1, HW), 1) % W_out
        m_left = (col != 0).astype(jnp.bfloat16)
        m_right = (col != W_out - 1).astype(jnp.bfloat16)

        zero = jnp.bfloat16(0)
        for i in range(KH * KW):
            kh, kw = divmod(i, KW)
            delta = (kh - (KH - 1) // 2) * W_out + (kw - (KW - 1) // 2)
            r0 = i * Cin
            lo = max(0, -delta)
            hi = min(HW, HW - delta)
            src = xb[:, lo + delta:hi + delta]
            if kw == 0:
                src = src * m_left[:, lo:hi]
            elif kw == KW - 1:
                src = src * m_right[:, lo:hi]
            if lo > 0:
                xcol_ref[r0:r0 + Cin, 0:lo] = jnp.full((Cin, lo), zero)
            if hi < HW:
                xcol_ref[r0:r0 + Cin, hi:HW] = jnp.full((Cin, HW - hi), zero)
            xcol_ref[r0:r0 + Cin, lo:hi] = src

        conv = jnp.dot(w_ref[...], xcol_ref[...],
                       preferred_element_type=jnp.float32)
        conv_ref[0] = conv.astype(jnp.bfloat16)
        s = jnp.sum(conv, axis=1, keepdims=True)
        s2 = jnp.sum(conv * conv, axis=1, keepdims=True)
        stats_ref[0] = jnp.concatenate([s, s2], axis=1)

    return _kernel


def _bn_relu_kernel(conv_ref, scale_ref, shift_ref, o_ref):
    y = conv_ref[0].astype(jnp.float32) * scale_ref[...] + shift_ref[...]
    o_ref[0] = jnp.maximum(y, 0.0)


@jax.jit
def _forward(x, w, gamma, beta):
    pad, eps = 1, 1e-3
    N, Cin, H, W = x.shape
    Cout, _, KH, KW = w.shape
    H_out = H + 2 * pad - KH + 1
    W_out = W + 2 * pad - KW + 1
    HW = H_out * W_out
    K = KH * KW * Cin

    w2 = jnp.transpose(w, (0, 2, 3, 1)).reshape(Cout, K).astype(jnp.bfloat16)
    x_flat = x.reshape(N, Cin, HW)

    cparams = pltpu.CompilerParams(
        dimension_semantics=("parallel",),
        vmem_limit_bytes=64 * 1024 * 1024,
    )

    conv, stats = pl.pallas_call(
        _make_conv_stats_kernel(KH, KW, W_out),
        grid=(N,),
        in_specs=[
            pl.BlockSpec((1, Cin, HW), lambda n: (n, 0, 0)),
            pl.BlockSpec((Cout, K), lambda n: (0, 0)),
        ],
        out_specs=(
            pl.BlockSpec((1, Cout, HW), lambda n: (n, 0, 0)),
            pl.BlockSpec((1, Cout, 2), lambda n: (n, 0, 0)),
        ),
        out_shape=(
            jax.ShapeDtypeStruct((N, Cout, HW), jnp.bfloat16),
            jax.ShapeDtypeStruct((N, Cout, 2), jnp.float32),
        ),
        scratch_shapes=[pltpu.VMEM((K, HW), jnp.bfloat16)],
        compiler_params=cparams,
    )(x_flat, w2)

    count = N * HW
    mean = stats[:, :, 0].sum(axis=0) / count
    var = stats[:, :, 1].sum(axis=0) / count - mean * mean
    scale = gamma * jax.lax.rsqrt(var + eps)
    shift = beta - mean * scale

    y_flat = pl.pallas_call(
        _bn_relu_kernel,
        grid=(N,),
        in_specs=[
            pl.BlockSpec((1, Cout, HW), lambda n: (n, 0, 0)),
            pl.BlockSpec((Cout, 1), lambda n: (0, 0)),
            pl.BlockSpec((Cout, 1), lambda n: (0, 0)),
        ],
        out_specs=pl.BlockSpec((1, Cout, HW), lambda n: (n, 0, 0)),
        out_shape=jax.ShapeDtypeStruct((N, Cout, HW), x.dtype),
        compiler_params=cparams,
    )(conv, scale.reshape(Cout, 1), shift.reshape(Cout, 1))

    return y_flat.reshape(N, Cout, H_out, W_out)


def kernel(x, w, gamma, beta):
    return _forward(x, w, gamma, beta)


# speedup vs baseline: 2.3004x; 1.1359x over previous
import functools
import math

import jax
import jax.numpy as jnp
from jax.experimental import pallas as pl
from jax.experimental.pallas import tpu as pltpu


def _make_conv_stats_kernel(KH, KW, W_out):
    def _kernel(x_ref, w_ref, conv_ref, stats_ref, xcol_ref):
        B, Cin, HW = x_ref.shape

        col = jax.lax.broadcasted_iota(jnp.int32, (1, HW), 1) % W_out
        m_left = (col != 0).astype(jnp.bfloat16)
        m_right = (col != W_out - 1).astype(jnp.bfloat16)

        zero = jnp.bfloat16(0)
        s_acc = jnp.zeros((x_ref.shape[1], 1), jnp.float32)
        s2_acc = jnp.zeros((x_ref.shape[1], 1), jnp.float32)
        for b in range(B):
            xb = x_ref[b].astype(jnp.bfloat16)
            for i in range(KH * KW):
                kh, kw = divmod(i, KW)
                delta = (kh - (KH - 1) // 2) * W_out + (kw - (KW - 1) // 2)
                r0 = i * Cin
                lo = max(0, -delta)
                hi = min(HW, HW - delta)
                src = xb[:, lo + delta:hi + delta]
                if kw == 0:
                    src = src * m_left[:, lo:hi]
                elif kw == KW - 1:
                    src = src * m_right[:, lo:hi]
                if lo > 0:
                    xcol_ref[r0:r0 + Cin, 0:lo] = jnp.full((Cin, lo), zero)
                if hi < HW:
                    xcol_ref[r0:r0 + Cin, hi:HW] = jnp.full((Cin, HW - hi), zero)
                xcol_ref[r0:r0 + Cin, lo:hi] = src

            conv = jnp.dot(w_ref[...], xcol_ref[...],
                           preferred_element_type=jnp.float32)
            conv_ref[b] = conv.astype(jnp.bfloat16)
            s_acc = s_acc + jnp.sum(conv, axis=1, keepdims=True)
            s2_acc = s2_acc + jnp.sum(conv * conv, axis=1, keepdims=True)
        stats_ref[0] = jnp.concatenate([s_acc, s2_acc], axis=1)

    return _kernel


def _bn_relu_kernel(conv_ref, scale_ref, shift_ref, o_ref):
    B = conv_ref.shape[0]
    for b in range(B):
        y = conv_ref[b].astype(jnp.float32) * scale_ref[...] + shift_ref[...]
        o_ref[b] = jnp.maximum(y, 0.0)


@jax.jit
def _forward(x, w, gamma, beta):
    pad, eps = 1, 1e-3
    N, Cin, H, W = x.shape
    Cout, _, KH, KW = w.shape
    H_out = H + 2 * pad - KH + 1
    W_out = W + 2 * pad - KW + 1
    HW = H_out * W_out
    K = KH * KW * Cin

    w2 = jnp.transpose(w, (0, 2, 3, 1)).reshape(Cout, K).astype(jnp.bfloat16)
    x_flat = x.reshape(N, Cin, HW)

    cparams = pltpu.CompilerParams(
        dimension_semantics=("parallel",),
        vmem_limit_bytes=64 * 1024 * 1024,
    )

    B1 = math.gcd(N, 4)
    B2 = math.gcd(N, 8)
    G1, G2 = N // B1, N // B2

    conv, stats = pl.pallas_call(
        _make_conv_stats_kernel(KH, KW, W_out),
        grid=(G1,),
        in_specs=[
            pl.BlockSpec((B1, Cin, HW), lambda n: (n, 0, 0)),
            pl.BlockSpec((Cout, K), lambda n: (0, 0)),
        ],
        out_specs=(
            pl.BlockSpec((B1, Cout, HW), lambda n: (n, 0, 0)),
            pl.BlockSpec((1, Cout, 2), lambda n: (n, 0, 0)),
        ),
        out_shape=(
            jax.ShapeDtypeStruct((N, Cout, HW), jnp.bfloat16),
            jax.ShapeDtypeStruct((G1, Cout, 2), jnp.float32),
        ),
        scratch_shapes=[pltpu.VMEM((K, HW), jnp.bfloat16)],
        compiler_params=cparams,
    )(x_flat, w2)

    count = N * HW
    mean = stats[:, :, 0].sum(axis=0) / count
    var = stats[:, :, 1].sum(axis=0) / count - mean * mean
    scale = gamma * jax.lax.rsqrt(var + eps)
    shift = beta - mean * scale

    y_flat = pl.pallas_call(
        _bn_relu_kernel,
        grid=(G2,),
        in_specs=[
            pl.BlockSpec((B2, Cout, HW), lambda n: (n, 0, 0)),
            pl.BlockSpec((Cout, 1), lambda n: (0, 0)),
            pl.BlockSpec((Cout, 1), lambda n: (0, 0)),
        ],
        out_specs=pl.BlockSpec((B2, Cout, HW), lambda n: (n, 0, 0)),
        out_shape=jax.ShapeDtypeStruct((N, Cout, HW), x.dtype),
        compiler_params=cparams,
    )(conv, scale.reshape(Cout, 1), shift.reshape(Cout, 1))

    return y_flat.reshape(N, Cout, H_out, W_out)


def kernel(x, w, gamma, beta):
    return _forward(x, w, gamma, beta)


# speedup vs baseline: 2.5261x; 1.0981x over previous
import functools
import math

import jax
import jax.numpy as jnp
from jax.experimental import pallas as pl
from jax.experimental.pallas import tpu as pltpu


def _shifted_tap(xb, delta, mask, HW):
    lo = max(0, -delta)
    hi = min(HW, HW - delta)
    src = xb[:, lo + delta:hi + delta]
    if mask is not None:
        src = src * mask[:, lo:hi]
    pieces = []
    if lo > 0:
        pieces.append(jnp.zeros((xb.shape[0], lo), jnp.bfloat16))
    pieces.append(src)
    if hi < HW:
        pieces.append(jnp.zeros((xb.shape[0], HW - hi), jnp.bfloat16))
    return pieces[0] if len(pieces) == 1 else jnp.concatenate(pieces, axis=1)


def _make_conv_stats_kernel(KH, KW, W_out):
    def _kernel(x_ref, w_ref, conv_ref, stats_ref):
        B, Cin, HW = x_ref.shape
        Cout = w_ref.shape[1]

        col = jax.lax.broadcasted_iota(jnp.int32, (1, HW), 1) % W_out
        m_left = (col != 0).astype(jnp.bfloat16)
        m_right = (col != W_out - 1).astype(jnp.bfloat16)
        masks = {0: m_left, KW - 1: m_right}

        s_acc = jnp.zeros((Cout, 1), jnp.float32)
        s2_acc = jnp.zeros((Cout, 1), jnp.float32)
        for b in range(B):
            xb = x_ref[b].astype(jnp.bfloat16)
            conv = jnp.zeros((Cout, HW), jnp.float32)
            for kh in range(KH):
                taps = [
                    _shifted_tap(
                        xb,
                        (kh - (KH - 1) // 2) * W_out + (kw - (KW - 1) // 2),
                        masks.get(kw), HW)
                    for kw in range(KW)
                ]
                g = jnp.concatenate(taps, axis=0)
                conv = conv + jnp.dot(w_ref[kh], g,
                                      preferred_element_type=jnp.float32)
            conv_ref[b] = conv.astype(jnp.bfloat16)
            s_acc = s_acc + jnp.sum(conv, axis=1, keepdims=True)
            s2_acc = s2_acc + jnp.sum(conv * conv, axis=1, keepdims=True)
        stats_ref[0] = jnp.concatenate([s_acc, s2_acc], axis=1)

    return _kernel


def _bn_relu_kernel(conv_ref, scale_ref, shift_ref, o_ref):
    B = conv_ref.shape[0]
    for b in range(B):
        y = conv_ref[b].astype(jnp.float32) * scale_ref[...] + shift_ref[...]
        o_ref[b] = jnp.maximum(y, 0.0)


@jax.jit
def _forward(x, w, gamma, beta):
    pad, eps = 1, 1e-3
    N, Cin, H, W = x.shape
    Cout, _, KH, KW = w.shape
    H_out = H + 2 * pad - KH + 1
    W_out = W + 2 * pad - KW + 1
    HW = H_out * W_out
    K = KH * KW * Cin

    w2 = (jnp.transpose(w, (2, 0, 3, 1))
          .reshape(KH, Cout, KW * Cin).astype(jnp.bfloat16))
    x_flat = x.reshape(N, Cin, HW)

    cparams = pltpu.CompilerParams(
        dimension_semantics=("parallel",),
        vmem_limit_bytes=64 * 1024 * 1024,
    )

    B1 = math.gcd(N, 4)
    B2 = math.gcd(N, 16)
    G1, G2 = N // B1, N // B2

    conv, stats = pl.pallas_call(
        _make_conv_stats_kernel(KH, KW, W_out),
        grid=(G1,),
        in_specs=[
            pl.BlockSpec((B1, Cin, HW), lambda n: (n, 0, 0)),
            pl.BlockSpec((KH, Cout, KW * Cin), lambda n: (0, 0, 0)),
        ],
        out_specs=(
            pl.BlockSpec((B1, Cout, HW), lambda n: (n, 0, 0)),
            pl.BlockSpec((1, Cout, 2), lambda n: (n, 0, 0)),
        ),
        out_shape=(
            jax.ShapeDtypeStruct((N, Cout, HW), jnp.bfloat16),
            jax.ShapeDtypeStruct((G1, Cout, 2), jnp.float32),
        ),
        compiler_params=cparams,
    )(x_flat, w2)

    count = N * HW
    mean = stats[:, :, 0].sum(axis=0) / count
    var = stats[:, :, 1].sum(axis=0) / count - mean * mean
    scale = gamma * jax.lax.rsqrt(var + eps)
    shift = beta - mean * scale

    y_flat = pl.pallas_call(
        _bn_relu_kernel,
        grid=(G2,),
        in_specs=[
            pl.BlockSpec((B2, Cout, HW), lambda n: (n, 0, 0)),
            pl.BlockSpec((Cout, 1), lambda n: (0, 0)),
            pl.BlockSpec((Cout, 1), lambda n: (0, 0)),
        ],
        out_specs=pl.BlockSpec((B2, Cout, HW), lambda n: (n, 0, 0)),
        out_shape=jax.ShapeDtypeStruct((N, Cout, HW), x.dtype),
        compiler_params=cparams,
    )(conv, scale.reshape(Cout, 1), shift.reshape(Cout, 1))

    return y_flat.reshape(N, Cout, H_out, W_out)


def kernel(x, w, gamma, beta):
    return _forward(x, w, gamma, beta)


# speedup vs baseline: 4.1770x; 1.6535x over previous
import functools
import math

import jax
import jax.numpy as jnp
from jax.experimental import pallas as pl
from jax.experimental.pallas import tpu as pltpu


def _make_conv_stats_kernel(KH, KW, W_out):
    def _kernel(x_ref, w_ref, conv_ref, stats_ref):
        B, HW, Cin = x_ref.shape
        Cout = w_ref.shape[1]
        KWC = KW * Cin

        row = jax.lax.broadcasted_iota(jnp.int32, (HW, 1), 0) % W_out
        m_first = (row != W_out - 1).astype(jnp.bfloat16)
        m_last = (row != 0).astype(jnp.bfloat16)

        s_acc = jnp.zeros((1, Cout), jnp.float32)
        s2_acc = jnp.zeros((1, Cout), jnp.float32)
        for b in range(B):
            xb = x_ref[b].astype(jnp.bfloat16)
            variants = []
            for kw in range(KW):
                v = xb
                if kw == 0:
                    v = v * m_first
                if kw == KW - 1:
                    v = v * m_last
                variants.append(v)

            conv = jnp.zeros((HW, Cout), jnp.float32)
            for kh in range(KH):
                taps = []
                for kw in range(KW):
                    delta = ((kh - (KH - 1) // 2) * W_out
                             + (kw - (KW - 1) // 2))
                    lo = max(0, -delta)
                    hi = min(HW, HW - delta)
                    src = variants[kw][lo + delta:hi + delta, :]
                    pieces = []
                    if lo > 0:
                        pieces.append(jnp.zeros((lo, Cin), jnp.bfloat16))
                    pieces.append(src)
                    if hi < HW:
                        pieces.append(jnp.zeros((HW - hi, Cin), jnp.bfloat16))
                    taps.append(pieces[0] if len(pieces) == 1
                                else jnp.concatenate(pieces, axis=0))
                g = jnp.concatenate(taps, axis=1)
                conv = conv + jnp.dot(g, w_ref[kh * KWC:(kh + 1) * KWC, :],
                                      preferred_element_type=jnp.float32)

            conv_ref[b] = conv.astype(jnp.bfloat16)
            s_acc = s_acc + jnp.sum(conv, axis=0, keepdims=True)
            s2_acc = s2_acc + jnp.sum(conv * conv, axis=0, keepdims=True)
        stats_ref[0] = jnp.concatenate([s_acc, s2_acc], axis=0)

    return _kernel


def _bn_relu_kernel(conv_ref, scale_ref, shift_ref, o_ref):
    B = conv_ref.shape[0]
    for b in range(B):
        y = conv_ref[b].astype(jnp.float32) * scale_ref[...] + shift_ref[...]
        o_ref[b] = jnp.maximum(y, 0.0)


@jax.jit
def _forward(x, w, gamma, beta):
    pad, eps = 1, 1e-3
    N, Cin, H, W = x.shape
    Cout, _, KH, KW = w.shape
    H_out = H + 2 * pad - KH + 1
    W_out = W + 2 * pad - KW + 1
    HW = H_out * W_out
    K = KH * KW * Cin

    xt = x.transpose(0, 2, 3, 1).reshape(N, HW, Cin)
    w2 = (jnp.transpose(w, (2, 3, 1, 0))
          .reshape(K, Cout).astype(jnp.bfloat16))

    cparams = pltpu.CompilerParams(
        dimension_semantics=("parallel",),
        vmem_limit_bytes=64 * 1024 * 1024,
    )

    B1 = math.gcd(N, 4)
    B2 = math.gcd(N, 4)
    G1, G2 = N // B1, N // B2

    conv, stats = pl.pallas_call(
        _make_conv_stats_kernel(KH, KW, W_out),
        grid=(G1,),
        in_specs=[
            pl.BlockSpec((B1, HW, Cin), lambda n: (n, 0, 0)),
            pl.BlockSpec((K, Cout), lambda n: (0, 0)),
        ],
        out_specs=(
            pl.BlockSpec((B1, HW, Cout), lambda n: (n, 0, 0)),
            pl.BlockSpec((1, 2, Cout), lambda n: (n, 0, 0)),
        ),
        out_shape=(
            jax.ShapeDtypeStruct((N, HW, Cout), jnp.bfloat16),
            jax.ShapeDtypeStruct((G1, 2, Cout), jnp.float32),
        ),
        compiler_params=cparams,
    )(xt, w2)

    count = N * HW
    tot = stats.sum(axis=0)
    mean = tot[0] / count
    var = tot[1] / count - mean * mean
    scale = gamma * jax.lax.rsqrt(var + eps)
    shift = beta - mean * scale

    y = pl.pallas_call(
        _bn_relu_kernel,
        grid=(G2,),
        in_specs=[
            pl.BlockSpec((B2, HW, Cout), lambda n: (n, 0, 0)),
            pl.BlockSpec((1, Cout), lambda n: (0, 0)),
            pl.BlockSpec((1, Cout), lambda n: (0, 0)),
        ],
        out_specs=pl.BlockSpec((B2, HW, Cout), lambda n: (n, 0, 0)),
        out_shape=jax.ShapeDtypeStruct((N, HW, Cout), x.dtype),
        compiler_params=cparams,
    )(conv, scale.reshape(1, Cout), shift.reshape(1, Cout))

    return y.reshape(N, H_out, W_out, Cout).transpose(0, 3, 1, 2)


def kernel(x, w, gamma, beta):
    return _forward(x, w, gamma, beta)


# speedup vs baseline: 4.1859x; 1.0021x over previous
import functools
import math

import jax
import jax.numpy as jnp
from jax.experimental import pallas as pl
from jax.experimental.pallas import tpu as pltpu


def _make_conv_stats_kernel(KH, KW, W_out):
    def _kernel(x_ref, w_ref, conv_ref, stats_ref):
        B, HW, Cin = x_ref.shape
        Cout = w_ref.shape[1]
        KWC = KW * Cin

        row = jax.lax.broadcasted_iota(jnp.int32, (HW, 1), 0) % W_out
        m_first = (row != W_out - 1).astype(jnp.bfloat16)
        m_last = (row != 0).astype(jnp.bfloat16)

        s_acc = jnp.zeros((1, Cout), jnp.float32)
        s2_acc = jnp.zeros((1, Cout), jnp.float32)
        for b in range(B):
            xb = x_ref[b].astype(jnp.bfloat16)
            variants = []
            for kw in range(KW):
                v = xb
                if kw == 0:
                    v = v * m_first
                if kw == KW - 1:
                    v = v * m_last
                variants.append(v)

            conv = jnp.zeros((HW, Cout), jnp.float32)
            for kh in range(KH):
                taps = []
                for kw in range(KW):
                    delta = ((kh - (KH - 1) // 2) * W_out
                             + (kw - (KW - 1) // 2))
                    lo = max(0, -delta)
                    hi = min(HW, HW - delta)
                    src = variants[kw][lo + delta:hi + delta, :]
                    pieces = []
                    if lo > 0:
                        pieces.append(jnp.zeros((lo, Cin), jnp.bfloat16))
                    pieces.append(src)
                    if hi < HW:
                        pieces.append(jnp.zeros((HW - hi, Cin), jnp.bfloat16))
                    taps.append(pieces[0] if len(pieces) == 1
                                else jnp.concatenate(pieces, axis=0))
                g = jnp.concatenate(taps, axis=1)
                conv = conv + jnp.dot(g, w_ref[kh * KWC:(kh + 1) * KWC, :],
                                      preferred_element_type=jnp.float32)

            conv_ref[b] = conv.astype(jnp.bfloat16)
            s_acc = s_acc + jnp.sum(conv, axis=0, keepdims=True)
            s2_acc = s2_acc + jnp.sum(conv * conv, axis=0, keepdims=True)
        stats_ref[0] = jnp.concatenate([s_acc, s2_acc], axis=0)

    return _kernel


def _bn_relu_kernel(conv_ref, scale_ref, shift_ref, o_ref):
    B = conv_ref.shape[0]
    for b in range(B):
        y = conv_ref[b].astype(jnp.float32) * scale_ref[...] + shift_ref[...]
        o_ref[b] = jnp.maximum(y, 0.0)


@jax.jit
def _forward(x, w, gamma, beta):
    pad, eps = 1, 1e-3
    N, Cin, H, W = x.shape
    Cout, _, KH, KW = w.shape
    H_out = H + 2 * pad - KH + 1
    W_out = W + 2 * pad - KW + 1
    HW = H_out * W_out
    K = KH * KW * Cin

    xt = x.transpose(0, 2, 3, 1).reshape(N, HW, Cin)
    w2 = (jnp.transpose(w, (2, 3, 1, 0))
          .reshape(K, Cout).astype(jnp.bfloat16))

    cparams = pltpu.CompilerParams(
        dimension_semantics=("parallel", "arbitrary"),
        vmem_limit_bytes=64 * 1024 * 1024,
    )

    B1 = math.gcd(N, 4)
    B2 = math.gcd(N, 4)
    G1, G2 = N // B1, N // B2
    NC = 2 if G1 % 2 == 0 and G2 % 2 == 0 else 1

    conv, stats = pl.pallas_call(
        _make_conv_stats_kernel(KH, KW, W_out),
        grid=(NC, G1 // NC),
        in_specs=[
            pl.BlockSpec((B1, HW, Cin), lambda c, g: (c * (G1 // NC) + g, 0, 0)),
            pl.BlockSpec((K, Cout), lambda c, g: (0, 0)),
        ],
        out_specs=(
            pl.BlockSpec((B1, HW, Cout), lambda c, g: (c * (G1 // NC) + g, 0, 0)),
            pl.BlockSpec((1, 2, Cout), lambda c, g: (c * (G1 // NC) + g, 0, 0)),
        ),
        out_shape=(
            jax.ShapeDtypeStruct((N, HW, Cout), jnp.bfloat16),
            jax.ShapeDtypeStruct((G1, 2, Cout), jnp.float32),
        ),
        compiler_params=cparams,
    )(xt, w2)

    count = N * HW
    tot = stats.sum(axis=0)
    mean = tot[0] / count
    var = tot[1] / count - mean * mean
    scale = gamma * jax.lax.rsqrt(var + eps)
    shift = beta - mean * scale

    y = pl.pallas_call(
        _bn_relu_kernel,
        grid=(NC, G2 // NC),
        in_specs=[
            pl.BlockSpec((B2, HW, Cout), lambda c, g: (c * (G2 // NC) + g, 0, 0)),
            pl.BlockSpec((1, Cout), lambda c, g: (0, 0)),
            pl.BlockSpec((1, Cout), lambda c, g: (0, 0)),
        ],
        out_specs=pl.BlockSpec((B2, HW, Cout), lambda c, g: (c * (G2 // NC) + g, 0, 0)),
        out_shape=jax.ShapeDtypeStruct((N, HW, Cout), x.dtype),
        compiler_params=cparams,
    )(conv, scale.reshape(1, Cout), shift.reshape(1, Cout))

    return y.reshape(N, H_out, W_out, Cout).transpose(0, 3, 1, 2)


def kernel(x, w, gamma, beta):
    return _forward(x, w, gamma, beta)
